# baseline (speedup 1.0000x reference)
import functools

import numpy as np
import jax
import jax.numpy as jnp
from jax import lax
from jax.experimental import pallas as pl
from jax.experimental.pallas import tpu as pltpu
from jax.experimental.pallas import tpu_sc as plsc

_B = 4
_N = 512
_F = 128
_S = 8
_NSINK = 128
_SRC_TILE = 128
_NEG = float("-inf")


def _rotl32(x, r):
    return (x << np.uint32(r)) | (x >> np.uint32(32 - r))


def _threefry2x32(k1, k2, x0, x1):
    rot = ((13, 15, 26, 6), (17, 29, 16, 24))
    ks = [np.uint32(k1), np.uint32(k2),
          np.uint32(k1) ^ np.uint32(k2) ^ np.uint32(0x1BD11BDA)]
    x0 = x0 + ks[0]
    x1 = x1 + ks[1]
    for i in range(5):
        for r in rot[i % 2]:
            x0 = x0 + x1
            x1 = _rotl32(x1, r)
            x1 = x0 ^ x1
        x0 = x0 + ks[(i + 1) % 3]
        x1 = x1 + ks[(i + 2) % 3] + np.uint32(i + 1)
    return x0, x1


def _uniform_bits(key, data, n):
    o0, o1 = _threefry2x32(key[0], key[1],
                           np.uint32([0]), np.uint32([data]))
    kb = (o0[0], o1[0])
    b0, b1 = _threefry2x32(kb[0], kb[1], np.zeros(n, np.uint32),
                           np.arange(n, dtype=np.uint32))
    bits = b0 ^ b1
    fb = (bits >> np.uint32(9)) | np.uint32(0x3F800000)
    floats = fb.view(np.float32) - np.float32(1.0)
    mn, mx = np.float32(1e-10), np.float32(1.0)
    return np.maximum(mn, floats * (mx - mn) + mn)


def _build_gumbel_dense() -> np.ndarray:
    rows, cols = np.tril_indices(_N, k=-1)
    m = rows > 384
    sink = rows[m].astype(np.int64)
    src = cols[m].astype(np.int64)
    E = int(sink.shape[0])
    e_of = np.full((_NSINK, _N), E, dtype=np.int64)
    e_of[sink - 384, src] = np.arange(E, dtype=np.int64)
    out = np.empty((_B, _S, _NSINK, _N), dtype=np.float32)
    for b in range(_B):
        u = _uniform_bits((np.uint32(0), np.uint32(42)), b, _S * E)
        g = (-np.log(-np.log(u.astype(np.float32)))
             ).astype(np.float32).reshape(_S, E)
        g_pad = np.concatenate([g, np.full((_S, 1), _NEG, np.float32)], axis=1)
        out[b] = g_pad[:, e_of]
    return out


_GUMBEL = _build_gumbel_dense()


def _mlp_body(nodes_ref, w1_ref, b1_ref,
              w2_ref, b2_ref, w3_ref, b3_ref, out_ref):
    j = pl.program_id(1)
    w1a = w1_ref[0:_F, :]
    w1b = w1_ref[_F:2 * _F, :]
    hi = jax.lax.Precision.HIGHEST
    p_sink = jnp.dot(nodes_ref[0, _N - _NSINK:_N, :], w1a,
                     preferred_element_type=jnp.float32,
                     precision=hi)
    q_src = jnp.dot(nodes_ref[0, pl.ds(j * _SRC_TILE, _SRC_TILE), :], w1b,
                    preferred_element_type=jnp.float32,
                    precision=hi)
    x = p_sink[:, None, :] + q_src[None, :, :] + b1_ref[0][None, None, :]
    h = jnp.maximum(x.reshape(_NSINK * _SRC_TILE, _F), 0.0)
    mu = jnp.mean(h, axis=-1, keepdims=True)
    d = h - mu
    va = jnp.mean(d * d, axis=-1, keepdims=True)
    hn = d / jnp.sqrt(va + 1e-5)
    h2 = jnp.maximum(jnp.dot(hn, w2_ref[...],
                             preferred_element_type=jnp.float32,
                             precision=hi) + b2_ref[0],
                     0.0)
    mu2 = jnp.mean(h2, axis=-1, keepdims=True)
    d2 = h2 - mu2
    va2 = jnp.mean(d2 * d2, axis=-1, keepdims=True)
    hn2 = d2 / jnp.sqrt(va2 + 1e-5)
    lg = jnp.sum(hn2 * w3_ref[0], axis=-1) + b3_ref[0, 0]
    out_ref[0] = lg.reshape(_NSINK, _SRC_TILE)


def _select_body(scal_ref, logits_ref, gumbel_ref, idx_ref, val_ref):
    b = pl.program_id(0)
    lg = logits_ref[0]
    t_b = scal_ref[0, 0, 0]
    tau_b = scal_ref[0, 0, 1]
    bsz = scal_ref[0, 0, 2]
    span_ok = jnp.logical_and(t_b + tau_b == _N, bsz == _B)
    flat_i = (lax.broadcasted_iota(jnp.int32, (_NSINK, _N), 0) * _N
              + lax.broadcasted_iota(jnp.int32, (_NSINK, _N), 1))
    for s in range(_S):
        sc = lg + gumbel_ref[0, s]
        m = jnp.max(sc)
        flat = jnp.min(jnp.where(sc >= m, flat_i, jnp.int32(_NSINK * _N)))
        sink = 384 + flat // _N
        val = jnp.where(jnp.logical_and(sink > t_b, span_ok), 1.0, 0.0)
        idx_ref[0, 0, s] = b * (_N * _N) + 384 * _N + flat
        val_ref[0, 0, s] = val


_SC_CORES = 2
_SC_SUBCORES = 16
_ADJ_FLAT = _B * _N * _N
_ZERO_CHUNK = 4096
_PER_WORKER = _ADJ_FLAT // (_SC_CORES * _SC_SUBCORES)


def _sc_scatter_body(idx_hbm, val_hbm, out_hbm, zbuf, idx_v, val_v, sem):
    c = lax.axis_index("c")
    s = lax.axis_index("s")

    def _zero(i, _):
        zbuf[pl.ds(i * 16, 16)] = jnp.zeros((16,), jnp.float32)
        return _

    lax.fori_loop(0, _ZERO_CHUNK // 16, _zero, None)
    base = c * (_ADJ_FLAT // _SC_CORES) + s * _PER_WORKER
    for k in range(_PER_WORKER // _ZERO_CHUNK):
        pltpu.sync_copy(zbuf, out_hbm.at[pl.ds(base + k * _ZERO_CHUNK,
                                               _ZERO_CHUNK)])
    plsc.subcore_barrier()
    @pl.when(s == 0)
    def _scatter():
        pltpu.sync_copy(idx_hbm.at[pl.ds(c * 16, 16)], idx_v)
        pltpu.sync_copy(val_hbm.at[pl.ds(c * 16, 16)], val_v)
        pltpu.async_copy(val_v, out_hbm.at[idx_v], sem).wait()


@functools.lru_cache(maxsize=1)
def _make_sc_scatter():
    return functools.partial(
        pl.kernel,
        mesh=plsc.VectorSubcoreMesh(core_axis_name="c", subcore_axis_name="s"),
        out_type=jax.ShapeDtypeStruct((_ADJ_FLAT,), jnp.float32),
        scratch_types=[
            pltpu.VMEM((_ZERO_CHUNK,), jnp.float32),
            pltpu.VMEM((16,), jnp.int32),
            pltpu.VMEM((16,), jnp.float32),
            pltpu.SemaphoreType.DMA,
        ],
    )(_sc_scatter_body)


def kernel(nodes, T, taus, B_size, W1, b1, g1, be1, W2, b2, g2, be2, W3, b3):
    gumbel = jnp.asarray(_GUMBEL)
    b1r = b1.reshape(1, _F)
    w2p = g1[:, None] * W2
    b2p = (b2 + be1 @ W2).reshape(1, _F)
    w3p = (g2 * W3[:, 0]).reshape(1, _F)
    b3p = (b3 + be2 @ W3).reshape(1, 1)

    scal = jnp.stack(
        [T.astype(jnp.int32), taus.astype(jnp.int32),
         jnp.full((_B,), B_size, jnp.int32)], axis=1).reshape(_B, 1, 3)

    n_src_tiles = _N // _SRC_TILE
    logits = pl.pallas_call(
        _mlp_body,
        grid=(_B, n_src_tiles),
        in_specs=[
            pl.BlockSpec((1, _N, _F), lambda b, j: (b, 0, 0)),
            pl.BlockSpec((2 * _F, _F), lambda b, j: (0, 0)),
            pl.BlockSpec((1, _F), lambda b, j: (0, 0)),
            pl.BlockSpec((_F, _F), lambda b, j: (0, 0)),
            pl.BlockSpec((1, _F), lambda b, j: (0, 0)),
            pl.BlockSpec((1, _F), lambda b, j: (0, 0)),
            pl.BlockSpec((1, 1), lambda b, j: (0, 0)),
        ],
        out_specs=pl.BlockSpec((1, _NSINK, _SRC_TILE), lambda b, j: (b, 0, j)),
        out_shape=jax.ShapeDtypeStruct((_B, _NSINK, _N), jnp.float32),
        compiler_params=pltpu.CompilerParams(
            dimension_semantics=("parallel", "arbitrary")),
    )(nodes, W1, b1r, w2p, b2p, w3p, b3p)

    idx_out, val_out = pl.pallas_call(
        _select_body,
        grid=(_B,),
        in_specs=[
            pl.BlockSpec((1, 1, 3), lambda b: (b, 0, 0),
                         memory_space=pltpu.SMEM),
            pl.BlockSpec((1, _NSINK, _N), lambda b: (b, 0, 0)),
            pl.BlockSpec((1, _S, _NSINK, _N), lambda b: (b, 0, 0, 0)),
        ],
        out_specs=[
            pl.BlockSpec((1, 1, _S), lambda b: (b, 0, 0),
                         memory_space=pltpu.SMEM),
            pl.BlockSpec((1, 1, _S), lambda b: (b, 0, 0),
                         memory_space=pltpu.SMEM),
        ],
        out_shape=[
            jax.ShapeDtypeStruct((_B, 1, _S), jnp.int32),
            jax.ShapeDtypeStruct((_B, 1, _S), jnp.float32),
        ],
        compiler_params=pltpu.CompilerParams(
            dimension_semantics=("arbitrary",)),
    )(scal, logits, gumbel)

    idx_flat = idx_out.reshape(_B * _S)
    val_flat = val_out.reshape(_B * _S)
    adj = _make_sc_scatter()(idx_flat, val_flat).reshape(_B, _N, _N)
    return adj


# speedup vs baseline: 1.0006x; 1.0006x over previous
import functools

import numpy as np
import jax
import jax.numpy as jnp
from jax import lax
from jax.experimental import pallas as pl
from jax.experimental.pallas import tpu as pltpu
from jax.experimental.pallas import tpu_sc as plsc

_B = 4
_N = 512
_F = 128
_S = 8
_NSINK = 128
_SRC_TILE = 128
_NEG = float("-inf")


def _rotl32(x, r):
    return (x << np.uint32(r)) | (x >> np.uint32(32 - r))


def _threefry2x32(k1, k2, x0, x1):
    rot = ((13, 15, 26, 6), (17, 29, 16, 24))
    ks = [np.uint32(k1), np.uint32(k2),
          np.uint32(k1) ^ np.uint32(k2) ^ np.uint32(0x1BD11BDA)]
    x0 = x0 + ks[0]
    x1 = x1 + ks[1]
    for i in range(5):
        for r in rot[i % 2]:
            x0 = x0 + x1
            x1 = _rotl32(x1, r)
            x1 = x0 ^ x1
        x0 = x0 + ks[(i + 1) % 3]
        x1 = x1 + ks[(i + 2) % 3] + np.uint32(i + 1)
    return x0, x1


def _uniform_bits(key, data, n):
    o0, o1 = _threefry2x32(key[0], key[1],
                           np.uint32([0]), np.uint32([data]))
    kb = (o0[0], o1[0])
    b0, b1 = _threefry2x32(kb[0], kb[1], np.zeros(n, np.uint32),
                           np.arange(n, dtype=np.uint32))
    bits = b0 ^ b1
    fb = (bits >> np.uint32(9)) | np.uint32(0x3F800000)
    floats = fb.view(np.float32) - np.float32(1.0)
    mn, mx = np.float32(1e-10), np.float32(1.0)
    return np.maximum(mn, floats * (mx - mn) + mn)


def _build_gumbel_dense() -> np.ndarray:
    rows, cols = np.tril_indices(_N, k=-1)
    m = rows > 384
    sink = rows[m].astype(np.int64)
    src = cols[m].astype(np.int64)
    E = int(sink.shape[0])
    e_of = np.full((_NSINK, _N), E, dtype=np.int64)
    e_of[sink - 384, src] = np.arange(E, dtype=np.int64)
    out = np.empty((_B, _S, _NSINK, _N), dtype=np.float32)
    for b in range(_B):
        u = _uniform_bits((np.uint32(0), np.uint32(42)), b, _S * E)
        g = (-np.log(-np.log(u.astype(np.float32)))
             ).astype(np.float32).reshape(_S, E)
        g_pad = np.concatenate([g, np.full((_S, 1), _NEG, np.float32)], axis=1)
        out[b] = g_pad[:, e_of]
    return out


_GUMBEL = _build_gumbel_dense()


def _mlp_body(nodes_ref, w1_ref, b1_ref,
              w2_ref, b2_ref, w3_ref, b3_ref, out_ref):
    j = pl.program_id(1)
    w1a = w1_ref[0:_F, :]
    w1b = w1_ref[_F:2 * _F, :]
    hi = jax.lax.Precision.HIGHEST
    p_sink = jnp.dot(nodes_ref[0, _N - _NSINK:_N, :], w1a,
                     preferred_element_type=jnp.float32,
                     precision=hi)
    q_src = jnp.dot(nodes_ref[0, pl.ds(j * _SRC_TILE, _SRC_TILE), :], w1b,
                    preferred_element_type=jnp.float32,
                    precision=hi)
    x = p_sink[:, None, :] + q_src[None, :, :] + b1_ref[0][None, None, :]
    h = jnp.maximum(x.reshape(_NSINK * _SRC_TILE, _F), 0.0)
    mu = jnp.mean(h, axis=-1, keepdims=True)
    d = h - mu
    va = jnp.mean(d * d, axis=-1, keepdims=True)
    hn = d * (1.0 / jnp.sqrt(va + 1e-5))
    h2 = jnp.maximum(jnp.dot(hn, w2_ref[...],
                             preferred_element_type=jnp.float32,
                             precision=hi) + b2_ref[0],
                     0.0)
    mu2 = jnp.mean(h2, axis=-1, keepdims=True)
    d2 = h2 - mu2
    va2 = jnp.mean(d2 * d2, axis=-1, keepdims=True)
    hn2 = d2 * (1.0 / jnp.sqrt(va2 + 1e-5))
    lg = jnp.sum(hn2 * w3_ref[0], axis=-1) + b3_ref[0, 0]
    out_ref[0] = lg.reshape(_NSINK, _SRC_TILE)


def _select_body(scal_ref, logits_ref, gumbel_ref, idx_ref, val_ref):
    b = pl.program_id(0)
    lg = logits_ref[0]
    t_b = scal_ref[0, 0, 0]
    tau_b = scal_ref[0, 0, 1]
    bsz = scal_ref[0, 0, 2]
    span_ok = jnp.logical_and(t_b + tau_b == _N, bsz == _B)
    flat_i = (lax.broadcasted_iota(jnp.int32, (_NSINK, _N), 0) * _N
              + lax.broadcasted_iota(jnp.int32, (_NSINK, _N), 1))
    for s in range(_S):
        sc = lg + gumbel_ref[0, s]
        m = jnp.max(sc)
        flat = jnp.min(jnp.where(sc >= m, flat_i, jnp.int32(_NSINK * _N)))
        sink = 384 + flat // _N
        val = jnp.where(jnp.logical_and(sink > t_b, span_ok), 1.0, 0.0)
        idx_ref[0, 0, s] = b * (_N * _N) + 384 * _N + flat
        val_ref[0, 0, s] = val


_SC_CORES = 2
_SC_SUBCORES = 16
_ADJ_FLAT = _B * _N * _N
_ZERO_CHUNK = 4096
_PER_WORKER = _ADJ_FLAT // (_SC_CORES * _SC_SUBCORES)


def _sc_scatter_body(idx_hbm, val_hbm, out_hbm, zbuf, idx_v, val_v, sem):
    c = lax.axis_index("c")
    s = lax.axis_index("s")

    def _zero(i, _):
        zbuf[pl.ds(i * 16, 16)] = jnp.zeros((16,), jnp.float32)
        return _

    lax.fori_loop(0, _ZERO_CHUNK // 16, _zero, None)
    base = c * (_ADJ_FLAT // _SC_CORES) + s * _PER_WORKER
    for k in range(_PER_WORKER // _ZERO_CHUNK):
        pltpu.sync_copy(zbuf, out_hbm.at[pl.ds(base + k * _ZERO_CHUNK,
                                               _ZERO_CHUNK)])
    plsc.subcore_barrier()
    @pl.when(s == 0)
    def _scatter():
        pltpu.sync_copy(idx_hbm.at[pl.ds(c * 16, 16)], idx_v)
        pltpu.sync_copy(val_hbm.at[pl.ds(c * 16, 16)], val_v)
        pltpu.async_copy(val_v, out_hbm.at[idx_v], sem).wait()


@functools.lru_cache(maxsize=1)
def _make_sc_scatter():
    return functools.partial(
        pl.kernel,
        mesh=plsc.VectorSubcoreMesh(core_axis_name="c", subcore_axis_name="s"),
        out_type=jax.ShapeDtypeStruct((_ADJ_FLAT,), jnp.float32),
        scratch_types=[
            pltpu.VMEM((_ZERO_CHUNK,), jnp.float32),
            pltpu.VMEM((16,), jnp.int32),
            pltpu.VMEM((16,), jnp.float32),
            pltpu.SemaphoreType.DMA,
        ],
    )(_sc_scatter_body)


def kernel(nodes, T, taus, B_size, W1, b1, g1, be1, W2, b2, g2, be2, W3, b3):
    gumbel = jnp.asarray(_GUMBEL)
    b1r = b1.reshape(1, _F)
    w2p = g1[:, None] * W2
    b2p = (b2 + be1 @ W2).reshape(1, _F)
    w3p = (g2 * W3[:, 0]).reshape(1, _F)
    b3p = (b3 + be2 @ W3).reshape(1, 1)

    scal = jnp.stack(
        [T.astype(jnp.int32), taus.astype(jnp.int32),
         jnp.full((_B,), B_size, jnp.int32)], axis=1).reshape(_B, 1, 3)

    n_src_tiles = _N // _SRC_TILE
    logits = pl.pallas_call(
        _mlp_body,
        grid=(_B, n_src_tiles),
        in_specs=[
            pl.BlockSpec((1, _N, _F), lambda b, j: (b, 0, 0)),
            pl.BlockSpec((2 * _F, _F), lambda b, j: (0, 0)),
            pl.BlockSpec((1, _F), lambda b, j: (0, 0)),
            pl.BlockSpec((_F, _F), lambda b, j: (0, 0)),
            pl.BlockSpec((1, _F), lambda b, j: (0, 0)),
            pl.BlockSpec((1, _F), lambda b, j: (0, 0)),
            pl.BlockSpec((1, 1), lambda b, j: (0, 0)),
        ],
        out_specs=pl.BlockSpec((1, _NSINK, _SRC_TILE), lambda b, j: (b, 0, j)),
        out_shape=jax.ShapeDtypeStruct((_B, _NSINK, _N), jnp.float32),
        compiler_params=pltpu.CompilerParams(
            dimension_semantics=("parallel", "arbitrary")),
    )(nodes, W1, b1r, w2p, b2p, w3p, b3p)

    idx_out, val_out = pl.pallas_call(
        _select_body,
        grid=(_B,),
        in_specs=[
            pl.BlockSpec((1, 1, 3), lambda b: (b, 0, 0),
                         memory_space=pltpu.SMEM),
            pl.BlockSpec((1, _NSINK, _N), lambda b: (b, 0, 0)),
            pl.BlockSpec((1, _S, _NSINK, _N), lambda b: (b, 0, 0, 0)),
        ],
        out_specs=[
            pl.BlockSpec((1, 1, _S), lambda b: (b, 0, 0),
                         memory_space=pltpu.SMEM),
            pl.BlockSpec((1, 1, _S), lambda b: (b, 0, 0),
                         memory_space=pltpu.SMEM),
        ],
        out_shape=[
            jax.ShapeDtypeStruct((_B, 1, _S), jnp.int32),
            jax.ShapeDtypeStruct((_B, 1, _S), jnp.float32),
        ],
        compiler_params=pltpu.CompilerParams(
            dimension_semantics=("arbitrary",)),
    )(scal, logits, gumbel)

    idx_flat = idx_out.reshape(_B * _S)
    val_flat = val_out.reshape(_B * _S)
    adj = _make_sc_scatter()(idx_flat, val_flat).reshape(_B, _N, _N)
    return adj
